# baseline (speedup 1.0000x reference)
import functools

import jax
import jax.numpy as jnp
import numpy as np
from jax import lax
from jax.experimental import pallas as pl
from jax.experimental.pallas import tpu as pltpu
from jax.experimental.pallas import tpu_sc as plsc

N = 16384
COLS = 64
NC = 2
NS = 16
NW = NC * NS
GB = 128
NGC = 6
CHUNK = NGC * GB
W = 256
ROWS_PER_TILE = N // NS
LANES = 16

_PERM = np.arange(COLS).reshape(COLS // 32, 2, 16).transpose(0, 2, 1).reshape(COLS)


def _sc_body(n_chunks, b_hbm, vals_hbm, rows_hbm, cols_hbm, out_hbm,
             bspm, acc, window, cols_v, rows_v, vals_v, gbufs, fidx, fidx1, cbuf, wbase,
             gsem0, gsem1, isem0, isem1):
    gsem = (gsem0, gsem1)
    isem = (isem0, isem1)
    cid = lax.axis_index("c")
    sid = lax.axis_index("s")
    wid = sid * NC + cid
    iota16 = lax.iota(jnp.int32, LANES)

    def idx_start(ci, slot):
        row0 = (wid * n_chunks + ci) * NGC
        pltpu.async_copy(cols_hbm.at[pl.ds(row0, NGC)], cols_v.at[slot], isem[slot])
        pltpu.async_copy(rows_hbm.at[pl.ds(row0, NGC)], rows_v.at[slot], isem[slot])
        pltpu.async_copy(vals_hbm.at[pl.ds(row0, NGC)], vals_v.at[slot], isem[slot])

    def idx_wait(slot):
        pltpu.make_async_copy(cols_hbm.at[pl.ds(0, NGC)], cols_v.at[slot], isem[slot]).wait()
        pltpu.make_async_copy(rows_hbm.at[pl.ds(0, NGC)], rows_v.at[slot], isem[slot]).wait()
        pltpu.make_async_copy(vals_hbm.at[pl.ds(0, NGC)], vals_v.at[slot], isem[slot]).wait()

    def gathers_start(slot, h, g):
        pltpu.async_copy(bspm.at[cols_v.at[slot, g]], gbufs.at[h], gsem[h])

    def gathers_wait(slot, h, g):
        pltpu.make_async_copy(bspm.at[cols_v.at[slot, g]], gbufs.at[h], gsem[h]).wait()

    def flush_window():
        wb = wbase[0]
        for fb in range(W // 128):
            @pl.loop(0, 128 // LANES)
            def _fill(t):
                fidx[pl.ds(t * LANES, LANES)] = (wb + fb * 128) + t * LANES + iota16

            pltpu.sync_copy(window.at[pl.ds(fb * 128, 128)], acc.at[fidx], add=True)

        @pl.loop(0, W)
        def _zero(i):
            for q in range(COLS // LANES):
                window[i, pl.ds(q * LANES, LANES)] = jnp.zeros((LANES,), jnp.float32)

    def accum(h, n_idx, r, v, wb):
        roff = r - wb
        ab0 = gbufs[h, n_idx, pl.ds(0, 2 * LANES)]
        ab1 = gbufs[h, n_idx, pl.ds(2 * LANES, 2 * LANES)]
        a0, a1 = plsc.unpack(ab0, format=plsc.PackFormat.INTERLEAVED)
        a2, a3 = plsc.unpack(ab1, format=plsc.PackFormat.INTERLEAVED)
        for q, aq in enumerate((a0, a1, a2, a3)):
            sl = pl.ds(q * LANES, LANES)
            window[roff, sl] = window[roff, sl] + aq * v

    def direct(h, n_idx, r, v):
        fidx1[pl.ds(0, LANES)] = jnp.full((LANES,), r, jnp.int32)
        ab0 = gbufs[h, n_idx, pl.ds(0, 2 * LANES)]
        ab1 = gbufs[h, n_idx, pl.ds(2 * LANES, 2 * LANES)]
        a0, a1 = plsc.unpack(ab0, format=plsc.PackFormat.INTERLEAVED)
        a2, a3 = plsc.unpack(ab1, format=plsc.PackFormat.INTERLEAVED)
        for q, aq in enumerate((a0, a1, a2, a3)):
            cbuf[0, pl.ds(q * LANES, LANES)] = aq * v
        pltpu.sync_copy(cbuf, acc.at[fidx1], add=True)

    def compute_group(slot, h, g):
        @pl.loop(0, GB // LANES)
        def _sub(sg):
            rv = rows_v[slot, g, pl.ds(sg * LANES, LANES)]
            vv = vals_v[slot, g, pl.ds(sg * LANES, LANES)]
            r_f = rv[0]
            r_l = rv[LANES - 1]

            @pl.when(r_l >= wbase[0] + W)
            def _():
                flush_window()
                wbase[0] = r_f

            wb = wbase[0]

            @pl.when(r_l < wb + W)
            def _fast():
                for i in range(LANES):
                    accum(h, sg * LANES + i, rv[i], vv[i], wb)

            @pl.when(r_l >= wb + W)
            def _wild():
                for i in range(LANES):
                    direct(h, sg * LANES + i, rv[i], vv[i])

    @pl.loop(0, W)
    def _zero_row(i):
        for q in range(COLS // LANES):
            window[i, pl.ds(q * LANES, LANES)] = jnp.zeros((LANES,), jnp.float32)

    @pl.loop(0, LANES)
    def _zero_cbuf(i):
        for q in range(COLS // LANES):
            cbuf[i, pl.ds(q * LANES, LANES)] = jnp.zeros((LANES,), jnp.float32)

    for k in range(ROWS_PER_TILE // W):
        pltpu.sync_copy(window, acc.at[pl.ds(sid * ROWS_PER_TILE + k * W, W)])

    @pl.when(sid == 0)
    def _():
        pltpu.sync_copy(window, acc.at[pl.ds(N, W)])

    pltpu.sync_copy(b_hbm.at[pl.ds(sid * ROWS_PER_TILE, ROWS_PER_TILE)],
                    bspm.at[pl.ds(sid * ROWS_PER_TILE, ROWS_PER_TILE)])
    plsc.subcore_barrier()

    idx_start(0, 0)
    idx_wait(0)
    rv0 = rows_v[0, 0, pl.ds(0, LANES)]
    wbase[0] = rv0[0]
    gathers_start(0, 0, 0)
    idx_start(1, 1)

    @pl.loop(0, n_chunks, step=2)
    def _cpair(ci0):
        for sc in range(2):
            ci = ci0 + sc

            @pl.loop(0, NGC, step=2)
            def _gpair(g0):
                for hh in range(2):
                    g = g0 + hh

                    @pl.when((g == NGC - 1) & (ci < n_chunks - 1))
                    def _():
                        idx_wait(1 - sc)

                    @pl.when(g < NGC - 1)
                    def _():
                        gathers_start(sc, 1 - hh, g + 1)

                    @pl.when((g == NGC - 1) & (ci < n_chunks - 1))
                    def _():
                        gathers_start(1 - sc, 1 - hh, 0)

                    @pl.when((g == 0) & (ci >= 1) & (ci < n_chunks - 1))
                    def _():
                        idx_start(ci + 1, 1 - sc)

                    gathers_wait(sc, hh, g)
                    compute_group(sc, hh, g)

    flush_window()
    plsc.subcore_barrier()
    pltpu.sync_copy(acc.at[pl.ds(sid * ROWS_PER_TILE, ROWS_PER_TILE)],
                    out_hbm.at[cid, pl.ds(sid * ROWS_PER_TILE, ROWS_PER_TILE)])


def _combine_body(p_ref, o_ref):
    o_ref[...] = p_ref[0] + p_ref[1]


def kernel(matrix_B, A_vals, A_rows, A_cols):
    nnz = A_vals.shape[0]
    per_w = ((nnz + NW * 2 * CHUNK - 1) // (NW * 2 * CHUNK)) * (2 * CHUNK)
    total = per_w * NW
    n_chunks = per_w // CHUNK
    pad = total - nnz

    b16 = matrix_B[:, _PERM].astype(jnp.bfloat16)
    cols = jnp.pad(A_cols.astype(jnp.int32), (0, pad)).reshape(total // GB, GB)
    rows = jnp.pad(A_rows.astype(jnp.int32), (0, pad),
                   constant_values=N - 1).reshape(total // GB, GB)
    vals = jnp.pad(A_vals, (0, pad)).reshape(total // GB, GB)

    mesh = plsc.VectorSubcoreMesh(core_axis_name="c", subcore_axis_name="s")
    partials = pl.kernel(
        functools.partial(_sc_body, n_chunks),
        out_type=jax.ShapeDtypeStruct((NC, N, COLS), jnp.float32),
        mesh=mesh,
        compiler_params=pltpu.CompilerParams(use_tc_tiling_on_sc=False,
                                             needs_layout_passes=False),
        scratch_types=[
            pltpu.VMEM_SHARED((N, COLS), jnp.bfloat16),
            pltpu.VMEM_SHARED((N + W, COLS), jnp.float32),
            pltpu.VMEM((W, COLS), jnp.float32),
            pltpu.VMEM((2, NGC, GB), jnp.int32),
            pltpu.VMEM((2, NGC, GB), jnp.int32),
            pltpu.VMEM((2, NGC, GB), jnp.float32),
            pltpu.VMEM((2, GB, COLS), jnp.bfloat16),
            pltpu.VMEM((128,), jnp.int32),
            pltpu.VMEM((LANES,), jnp.int32),
            pltpu.VMEM((LANES, COLS), jnp.float32),
            pltpu.SMEM((8,), jnp.int32),
            pltpu.SemaphoreType.DMA,
            pltpu.SemaphoreType.DMA,
            pltpu.SemaphoreType.DMA,
            pltpu.SemaphoreType.DMA,
        ],
    )(b16, vals, rows, cols)

    out = pl.pallas_call(
        _combine_body,
        out_shape=jax.ShapeDtypeStruct((N, COLS), jnp.float32),
        grid=(N // 1024,),
        in_specs=[pl.BlockSpec((NC, 1024, COLS), lambda i: (0, i, 0))],
        out_specs=pl.BlockSpec((1024, COLS), lambda i: (i, 0)),
    )(partials)
    return out


# speedup vs baseline: 1.4487x; 1.4487x over previous
import functools

import jax
import jax.numpy as jnp
import numpy as np
from jax import lax
from jax.experimental import pallas as pl
from jax.experimental.pallas import tpu as pltpu
from jax.experimental.pallas import tpu_sc as plsc

N = 16384
COLS = 64
NC = 2
NS = 16
NW = NC * NS
GB = 128
NGC = 6
CHUNK = NGC * GB
W = 256
ROWS_PER_TILE = N // NS
LANES = 16

_PERM = np.arange(COLS).reshape(COLS // 32, 2, 16).transpose(0, 2, 1).reshape(COLS)


def _sc_body(n_chunks, b_hbm, vals_hbm, rows_hbm, cols_hbm, out_hbm,
             bspm, acc, window, cols_v, rows_v, vals_v, gbufs, fidx, fidx1, cbuf, wbase,
             gsem0, gsem1, isem0, isem1):
    gsem = (gsem0, gsem1)
    isem = (isem0, isem1)
    cid = lax.axis_index("c")
    sid = lax.axis_index("s")
    wid = sid * NC + cid
    iota16 = lax.iota(jnp.int32, LANES)

    def idx_start(ci, slot):
        row0 = (wid * n_chunks + ci) * NGC
        pltpu.async_copy(cols_hbm.at[pl.ds(row0, NGC)], cols_v.at[slot], isem[slot])
        pltpu.async_copy(rows_hbm.at[pl.ds(row0, NGC)], rows_v.at[slot], isem[slot])
        pltpu.async_copy(vals_hbm.at[pl.ds(row0, NGC)], vals_v.at[slot], isem[slot])

    def idx_wait(slot):
        pltpu.make_async_copy(cols_hbm.at[pl.ds(0, NGC)], cols_v.at[slot], isem[slot]).wait()
        pltpu.make_async_copy(rows_hbm.at[pl.ds(0, NGC)], rows_v.at[slot], isem[slot]).wait()
        pltpu.make_async_copy(vals_hbm.at[pl.ds(0, NGC)], vals_v.at[slot], isem[slot]).wait()

    def gathers_start(slot, h, g):
        pltpu.async_copy(bspm.at[cols_v.at[slot, g]], gbufs.at[h], gsem[h])

    def gathers_wait(slot, h, g):
        pltpu.make_async_copy(bspm.at[cols_v.at[slot, g]], gbufs.at[h], gsem[h]).wait()

    def flush_window():
        wb = wbase[0]
        for fb in range(W // 128):
            @pl.loop(0, 128 // LANES)
            def _fill(t):
                fidx[pl.ds(t * LANES, LANES)] = (wb + fb * 128) + t * LANES + iota16

            pltpu.sync_copy(window.at[pl.ds(fb * 128, 128)], acc.at[fidx], add=True)

        @pl.loop(0, W)
        def _zero(i):
            for q in range(COLS // LANES):
                window[i, pl.ds(q * LANES, LANES)] = jnp.zeros((LANES,), jnp.float32)

    def accum(h, n_idx, r, v, wb):
        roff = r - wb
        ab0 = gbufs[h, n_idx, pl.ds(0, 2 * LANES)]
        ab1 = gbufs[h, n_idx, pl.ds(2 * LANES, 2 * LANES)]
        a0, a1 = plsc.unpack(ab0, format=plsc.PackFormat.INTERLEAVED)
        a2, a3 = plsc.unpack(ab1, format=plsc.PackFormat.INTERLEAVED)
        for q, aq in enumerate((a0, a1, a2, a3)):
            sl = pl.ds(q * LANES, LANES)
            window[roff, sl] = window[roff, sl] + aq * v

    def direct(h, n_idx, r, v):
        fidx1[pl.ds(0, LANES)] = jnp.full((LANES,), r, jnp.int32)
        ab0 = gbufs[h, n_idx, pl.ds(0, 2 * LANES)]
        ab1 = gbufs[h, n_idx, pl.ds(2 * LANES, 2 * LANES)]
        a0, a1 = plsc.unpack(ab0, format=plsc.PackFormat.INTERLEAVED)
        a2, a3 = plsc.unpack(ab1, format=plsc.PackFormat.INTERLEAVED)
        for q, aq in enumerate((a0, a1, a2, a3)):
            cbuf[0, pl.ds(q * LANES, LANES)] = aq * v
        pltpu.sync_copy(cbuf, acc.at[fidx1], add=True)

    def unpack4(h, n_idx):
        ab0 = gbufs[h, n_idx, pl.ds(0, 2 * LANES)]
        ab1 = gbufs[h, n_idx, pl.ds(2 * LANES, 2 * LANES)]
        a0, a1 = plsc.unpack(ab0, format=plsc.PackFormat.INTERLEAVED)
        a2, a3 = plsc.unpack(ab1, format=plsc.PackFormat.INTERLEAVED)
        return (a0, a1, a2, a3)

    def compute_group(slot, h, g):
        @pl.loop(0, GB // LANES)
        def _sub(sg):
            rv = rows_v[slot, g, pl.ds(sg * LANES, LANES)]
            vv = vals_v[slot, g, pl.ds(sg * LANES, LANES)]
            r_f = rv[0]
            r_l = rv[LANES - 1]

            @pl.when(r_l >= wbase[0] + W)
            def _():
                flush_window()
                wbase[0] = r_f

            wb = wbase[0]

            @pl.when(r_f == r_l)
            def _run():
                s = [None] * (COLS // LANES)
                for i in range(LANES):
                    aq = unpack4(h, sg * LANES + i)
                    v = vv[i]
                    for q in range(COLS // LANES):
                        t = aq[q] * v
                        s[q] = t if s[q] is None else s[q] + t
                roff = r_f - wb
                for q in range(COLS // LANES):
                    sl = pl.ds(q * LANES, LANES)
                    window[roff, sl] = window[roff, sl] + s[q]

            @pl.when((r_l < wb + W) & (r_f != r_l))
            def _fast():
                for i in range(LANES):
                    accum(h, sg * LANES + i, rv[i], vv[i], wb)

            @pl.when(r_l >= wb + W)
            def _wild():
                for i in range(LANES):
                    direct(h, sg * LANES + i, rv[i], vv[i])

    @pl.loop(0, W)
    def _zero_row(i):
        for q in range(COLS // LANES):
            window[i, pl.ds(q * LANES, LANES)] = jnp.zeros((LANES,), jnp.float32)

    @pl.loop(0, LANES)
    def _zero_cbuf(i):
        for q in range(COLS // LANES):
            cbuf[i, pl.ds(q * LANES, LANES)] = jnp.zeros((LANES,), jnp.float32)

    for k in range(ROWS_PER_TILE // W):
        pltpu.sync_copy(window, acc.at[pl.ds(sid * ROWS_PER_TILE + k * W, W)])

    @pl.when(sid == 0)
    def _():
        pltpu.sync_copy(window, acc.at[pl.ds(N, W)])

    pltpu.sync_copy(b_hbm.at[pl.ds(sid * ROWS_PER_TILE, ROWS_PER_TILE)],
                    bspm.at[pl.ds(sid * ROWS_PER_TILE, ROWS_PER_TILE)])
    plsc.subcore_barrier()

    idx_start(0, 0)
    idx_wait(0)
    rv0 = rows_v[0, 0, pl.ds(0, LANES)]
    wbase[0] = rv0[0]
    gathers_start(0, 0, 0)
    idx_start(1, 1)

    @pl.loop(0, n_chunks, step=2)
    def _cpair(ci0):
        for sc in range(2):
            ci = ci0 + sc

            @pl.loop(0, NGC, step=2)
            def _gpair(g0):
                for hh in range(2):
                    g = g0 + hh

                    @pl.when((g == NGC - 1) & (ci < n_chunks - 1))
                    def _():
                        idx_wait(1 - sc)

                    @pl.when(g < NGC - 1)
                    def _():
                        gathers_start(sc, 1 - hh, g + 1)

                    @pl.when((g == NGC - 1) & (ci < n_chunks - 1))
                    def _():
                        gathers_start(1 - sc, 1 - hh, 0)

                    @pl.when((g == 0) & (ci >= 1) & (ci < n_chunks - 1))
                    def _():
                        idx_start(ci + 1, 1 - sc)

                    gathers_wait(sc, hh, g)
                    compute_group(sc, hh, g)

    flush_window()
    plsc.subcore_barrier()
    pltpu.sync_copy(acc.at[pl.ds(sid * ROWS_PER_TILE, ROWS_PER_TILE)],
                    out_hbm.at[cid, pl.ds(sid * ROWS_PER_TILE, ROWS_PER_TILE)])


def _combine_body(p_ref, o_ref):
    o_ref[...] = p_ref[0] + p_ref[1]


def kernel(matrix_B, A_vals, A_rows, A_cols):
    nnz = A_vals.shape[0]
    per_w = ((nnz + NW * 2 * CHUNK - 1) // (NW * 2 * CHUNK)) * (2 * CHUNK)
    total = per_w * NW
    n_chunks = per_w // CHUNK
    pad = total - nnz

    b16 = matrix_B[:, _PERM].astype(jnp.bfloat16)
    cols = jnp.pad(A_cols.astype(jnp.int32), (0, pad)).reshape(total // GB, GB)
    rows = jnp.pad(A_rows.astype(jnp.int32), (0, pad),
                   constant_values=N - 1).reshape(total // GB, GB)
    vals = jnp.pad(A_vals, (0, pad)).reshape(total // GB, GB)

    mesh = plsc.VectorSubcoreMesh(core_axis_name="c", subcore_axis_name="s")
    partials = pl.kernel(
        functools.partial(_sc_body, n_chunks),
        out_type=jax.ShapeDtypeStruct((NC, N, COLS), jnp.float32),
        mesh=mesh,
        compiler_params=pltpu.CompilerParams(use_tc_tiling_on_sc=False,
                                             needs_layout_passes=False),
        scratch_types=[
            pltpu.VMEM_SHARED((N, COLS), jnp.bfloat16),
            pltpu.VMEM_SHARED((N + W, COLS), jnp.float32),
            pltpu.VMEM((W, COLS), jnp.float32),
            pltpu.VMEM((2, NGC, GB), jnp.int32),
            pltpu.VMEM((2, NGC, GB), jnp.int32),
            pltpu.VMEM((2, NGC, GB), jnp.float32),
            pltpu.VMEM((2, GB, COLS), jnp.bfloat16),
            pltpu.VMEM((128,), jnp.int32),
            pltpu.VMEM((LANES,), jnp.int32),
            pltpu.VMEM((LANES, COLS), jnp.float32),
            pltpu.SMEM((8,), jnp.int32),
            pltpu.SemaphoreType.DMA,
            pltpu.SemaphoreType.DMA,
            pltpu.SemaphoreType.DMA,
            pltpu.SemaphoreType.DMA,
        ],
    )(b16, vals, rows, cols)

    out = pl.pallas_call(
        _combine_body,
        out_shape=jax.ShapeDtypeStruct((N, COLS), jnp.float32),
        grid=(N // 1024,),
        in_specs=[pl.BlockSpec((NC, 1024, COLS), lambda i: (0, i, 0))],
        out_specs=pl.BlockSpec((1024, COLS), lambda i: (i, 0)),
    )(partials)
    return out


# speedup vs baseline: 1.5493x; 1.0694x over previous
import functools

import jax
import jax.numpy as jnp
import numpy as np
from jax import lax
from jax.experimental import pallas as pl
from jax.experimental.pallas import tpu as pltpu
from jax.experimental.pallas import tpu_sc as plsc

N = 16384
COLS = 64
NC = 2
NS = 16
NW = NC * NS
GB = 256
NGC = 4
CHUNK = NGC * GB
W = 128
ROWS_PER_TILE = N // NS
LANES = 16

_PERM = np.arange(COLS).reshape(COLS // 32, 2, 16).transpose(0, 2, 1).reshape(COLS)


def _sc_body(n_chunks, b_hbm, vals_hbm, rows_hbm, cols_hbm, out_hbm,
             bspm, acc, window, cols_v, rows_v, vals_v, gbufs, fidx, fidx1, cbuf, wbase,
             gsem0, gsem1, isem0, isem1):
    gsem = (gsem0, gsem1)
    isem = (isem0, isem1)
    cid = lax.axis_index("c")
    sid = lax.axis_index("s")
    wid = sid * NC + cid
    iota16 = lax.iota(jnp.int32, LANES)

    def idx_start(ci, slot):
        row0 = (wid * n_chunks + ci) * NGC
        pltpu.async_copy(cols_hbm.at[pl.ds(row0, NGC)], cols_v.at[slot], isem[slot])
        pltpu.async_copy(rows_hbm.at[pl.ds(row0, NGC)], rows_v.at[slot], isem[slot])
        pltpu.async_copy(vals_hbm.at[pl.ds(row0, NGC)], vals_v.at[slot], isem[slot])

    def idx_wait(slot):
        pltpu.make_async_copy(cols_hbm.at[pl.ds(0, NGC)], cols_v.at[slot], isem[slot]).wait()
        pltpu.make_async_copy(rows_hbm.at[pl.ds(0, NGC)], rows_v.at[slot], isem[slot]).wait()
        pltpu.make_async_copy(vals_hbm.at[pl.ds(0, NGC)], vals_v.at[slot], isem[slot]).wait()

    def gathers_start(slot, h, g):
        pltpu.async_copy(bspm.at[cols_v.at[slot, g]], gbufs.at[h], gsem[h])

    def gathers_wait(slot, h, g):
        pltpu.make_async_copy(bspm.at[cols_v.at[slot, g]], gbufs.at[h], gsem[h]).wait()

    def flush_window():
        wb = wbase[0]
        for fb in range(W // 128):
            @pl.loop(0, 128 // LANES)
            def _fill(t):
                fidx[pl.ds(t * LANES, LANES)] = (wb + fb * 128) + t * LANES + iota16

            pltpu.sync_copy(window.at[pl.ds(fb * 128, 128)], acc.at[fidx], add=True)

        @pl.loop(0, W)
        def _zero(i):
            for q in range(COLS // LANES):
                window[i, pl.ds(q * LANES, LANES)] = jnp.zeros((LANES,), jnp.float32)

    def accum(h, n_idx, r, v, wb):
        roff = r - wb
        ab0 = gbufs[h, n_idx, pl.ds(0, 2 * LANES)]
        ab1 = gbufs[h, n_idx, pl.ds(2 * LANES, 2 * LANES)]
        a0, a1 = plsc.unpack(ab0, format=plsc.PackFormat.INTERLEAVED)
        a2, a3 = plsc.unpack(ab1, format=plsc.PackFormat.INTERLEAVED)
        for q, aq in enumerate((a0, a1, a2, a3)):
            sl = pl.ds(q * LANES, LANES)
            window[roff, sl] = window[roff, sl] + aq * v

    def direct(h, n_idx, r, v):
        fidx1[pl.ds(0, LANES)] = jnp.full((LANES,), r, jnp.int32)
        ab0 = gbufs[h, n_idx, pl.ds(0, 2 * LANES)]
        ab1 = gbufs[h, n_idx, pl.ds(2 * LANES, 2 * LANES)]
        a0, a1 = plsc.unpack(ab0, format=plsc.PackFormat.INTERLEAVED)
        a2, a3 = plsc.unpack(ab1, format=plsc.PackFormat.INTERLEAVED)
        for q, aq in enumerate((a0, a1, a2, a3)):
            cbuf[0, pl.ds(q * LANES, LANES)] = aq * v
        pltpu.sync_copy(cbuf, acc.at[fidx1], add=True)

    def unpack4(h, n_idx):
        ab0 = gbufs[h, n_idx, pl.ds(0, 2 * LANES)]
        ab1 = gbufs[h, n_idx, pl.ds(2 * LANES, 2 * LANES)]
        a0, a1 = plsc.unpack(ab0, format=plsc.PackFormat.INTERLEAVED)
        a2, a3 = plsc.unpack(ab1, format=plsc.PackFormat.INTERLEAVED)
        return (a0, a1, a2, a3)

    def compute_group(slot, h, g):
        @pl.loop(0, GB // LANES)
        def _sub(sg):
            rv = rows_v[slot, g, pl.ds(sg * LANES, LANES)]
            vv = vals_v[slot, g, pl.ds(sg * LANES, LANES)]
            r_f = rv[0]
            r_l = rv[LANES - 1]

            @pl.when(r_l >= wbase[0] + W)
            def _():
                flush_window()
                wbase[0] = r_f

            wb = wbase[0]

            @pl.when(r_f == r_l)
            def _run():
                s = [None] * (COLS // LANES)
                for i in range(LANES):
                    aq = unpack4(h, sg * LANES + i)
                    v = vv[i]
                    for q in range(COLS // LANES):
                        t = aq[q] * v
                        s[q] = t if s[q] is None else s[q] + t
                roff = r_f - wb
                for q in range(COLS // LANES):
                    sl = pl.ds(q * LANES, LANES)
                    window[roff, sl] = window[roff, sl] + s[q]

            @pl.when((r_l < wb + W) & (r_f != r_l))
            def _fast():
                for i in range(LANES):
                    accum(h, sg * LANES + i, rv[i], vv[i], wb)

            @pl.when(r_l >= wb + W)
            def _wild():
                for i in range(LANES):
                    direct(h, sg * LANES + i, rv[i], vv[i])

    @pl.loop(0, W)
    def _zero_row(i):
        for q in range(COLS // LANES):
            window[i, pl.ds(q * LANES, LANES)] = jnp.zeros((LANES,), jnp.float32)

    @pl.loop(0, LANES)
    def _zero_cbuf(i):
        for q in range(COLS // LANES):
            cbuf[i, pl.ds(q * LANES, LANES)] = jnp.zeros((LANES,), jnp.float32)

    for k in range(ROWS_PER_TILE // W):
        pltpu.sync_copy(window, acc.at[pl.ds(sid * ROWS_PER_TILE + k * W, W)])

    @pl.when(sid == 0)
    def _():
        pltpu.sync_copy(window, acc.at[pl.ds(N, W)])

    pltpu.sync_copy(b_hbm.at[pl.ds(sid * ROWS_PER_TILE, ROWS_PER_TILE)],
                    bspm.at[pl.ds(sid * ROWS_PER_TILE, ROWS_PER_TILE)])
    plsc.subcore_barrier()

    idx_start(0, 0)
    idx_wait(0)
    rv0 = rows_v[0, 0, pl.ds(0, LANES)]
    wbase[0] = rv0[0]
    gathers_start(0, 0, 0)
    idx_start(1, 1)

    @pl.loop(0, n_chunks, step=2)
    def _cpair(ci0):
        for sc in range(2):
            ci = ci0 + sc

            @pl.loop(0, NGC, step=2)
            def _gpair(g0):
                for hh in range(2):
                    g = g0 + hh

                    @pl.when((g == NGC - 1) & (ci < n_chunks - 1))
                    def _():
                        idx_wait(1 - sc)

                    @pl.when(g < NGC - 1)
                    def _():
                        gathers_start(sc, 1 - hh, g + 1)

                    @pl.when((g == NGC - 1) & (ci < n_chunks - 1))
                    def _():
                        gathers_start(1 - sc, 1 - hh, 0)

                    @pl.when((g == 0) & (ci >= 1) & (ci < n_chunks - 1))
                    def _():
                        idx_start(ci + 1, 1 - sc)

                    gathers_wait(sc, hh, g)
                    compute_group(sc, hh, g)

    flush_window()
    plsc.subcore_barrier()
    pltpu.sync_copy(acc.at[pl.ds(sid * ROWS_PER_TILE, ROWS_PER_TILE)],
                    out_hbm.at[cid, pl.ds(sid * ROWS_PER_TILE, ROWS_PER_TILE)])


def _combine_body(p_ref, o_ref):
    o_ref[...] = p_ref[0] + p_ref[1]


def kernel(matrix_B, A_vals, A_rows, A_cols):
    nnz = A_vals.shape[0]
    per_w = ((nnz + NW * 2 * CHUNK - 1) // (NW * 2 * CHUNK)) * (2 * CHUNK)
    total = per_w * NW
    n_chunks = per_w // CHUNK
    pad = total - nnz

    b16 = matrix_B[:, _PERM].astype(jnp.bfloat16)
    cols = jnp.pad(A_cols.astype(jnp.int32), (0, pad)).reshape(total // GB, GB)
    rows = jnp.pad(A_rows.astype(jnp.int32), (0, pad),
                   constant_values=N - 1).reshape(total // GB, GB)
    vals = jnp.pad(A_vals, (0, pad)).reshape(total // GB, GB)

    mesh = plsc.VectorSubcoreMesh(core_axis_name="c", subcore_axis_name="s")
    partials = pl.kernel(
        functools.partial(_sc_body, n_chunks),
        out_type=jax.ShapeDtypeStruct((NC, N, COLS), jnp.float32),
        mesh=mesh,
        compiler_params=pltpu.CompilerParams(use_tc_tiling_on_sc=False,
                                             needs_layout_passes=False),
        scratch_types=[
            pltpu.VMEM_SHARED((N, COLS), jnp.bfloat16),
            pltpu.VMEM_SHARED((N + W, COLS), jnp.float32),
            pltpu.VMEM((W, COLS), jnp.float32),
            pltpu.VMEM((2, NGC, GB), jnp.int32),
            pltpu.VMEM((2, NGC, GB), jnp.int32),
            pltpu.VMEM((2, NGC, GB), jnp.float32),
            pltpu.VMEM((2, GB, COLS), jnp.bfloat16),
            pltpu.VMEM((128,), jnp.int32),
            pltpu.VMEM((LANES,), jnp.int32),
            pltpu.VMEM((LANES, COLS), jnp.float32),
            pltpu.SMEM((8,), jnp.int32),
            pltpu.SemaphoreType.DMA,
            pltpu.SemaphoreType.DMA,
            pltpu.SemaphoreType.DMA,
            pltpu.SemaphoreType.DMA,
        ],
    )(b16, vals, rows, cols)

    out = pl.pallas_call(
        _combine_body,
        out_shape=jax.ShapeDtypeStruct((N, COLS), jnp.float32),
        grid=(N // 1024,),
        in_specs=[pl.BlockSpec((NC, 1024, COLS), lambda i: (0, i, 0))],
        out_specs=pl.BlockSpec((1024, COLS), lambda i: (i, 0)),
    )(partials)
    return out


# speedup vs baseline: 1.7098x; 1.1036x over previous
import functools

import jax
import jax.numpy as jnp
import numpy as np
from jax import lax
from jax.experimental import pallas as pl
from jax.experimental.pallas import tpu as pltpu
from jax.experimental.pallas import tpu_sc as plsc

N = 16384
COLS = 64
NC = 2
NS = 16
NW = NC * NS
GB = 256
NGC = 4
CHUNK = NGC * GB
W = 128
ROWS_PER_TILE = N // NS
LANES = 16

_PERM = np.arange(COLS).reshape(COLS // 32, 2, 16).transpose(0, 2, 1).reshape(COLS)


def _sc_body(n_chunks, b_hbm, vals_hbm, rows_hbm, cols_hbm, out_hbm,
             bspm, acc, window, cols_v, rows_v, vals_v, gbufs, fidx, fidx1, cbuf, wbase,
             gsem0, gsem1, isem0, isem1):
    gsem = (gsem0, gsem1)
    isem = (isem0, isem1)
    cid = lax.axis_index("c")
    sid = lax.axis_index("s")
    wid = sid * NC + cid
    iota16 = lax.iota(jnp.int32, LANES)

    def idx_start(ci, slot):
        row0 = (wid * n_chunks + ci) * NGC
        pltpu.async_copy(cols_hbm.at[pl.ds(row0, NGC)], cols_v.at[slot], isem[slot])
        pltpu.async_copy(rows_hbm.at[pl.ds(row0, NGC)], rows_v.at[slot], isem[slot])
        pltpu.async_copy(vals_hbm.at[pl.ds(row0, NGC)], vals_v.at[slot], isem[slot])

    def idx_wait(slot):
        pltpu.make_async_copy(cols_hbm.at[pl.ds(0, NGC)], cols_v.at[slot], isem[slot]).wait()
        pltpu.make_async_copy(rows_hbm.at[pl.ds(0, NGC)], rows_v.at[slot], isem[slot]).wait()
        pltpu.make_async_copy(vals_hbm.at[pl.ds(0, NGC)], vals_v.at[slot], isem[slot]).wait()

    def gathers_start(slot, h, g):
        pltpu.async_copy(bspm.at[cols_v.at[slot, g]], gbufs.at[h], gsem[h])

    def gathers_wait(slot, h, g):
        pltpu.make_async_copy(bspm.at[cols_v.at[slot, g]], gbufs.at[h], gsem[h]).wait()

    def flush_window():
        wb = wbase[0]
        for fb in range(W // 128):
            @pl.loop(0, 128 // LANES)
            def _fill(t):
                fidx[pl.ds(t * LANES, LANES)] = (wb + fb * 128) + t * LANES + iota16

            pltpu.sync_copy(window.at[pl.ds(fb * 128, 128)], acc.at[fidx], add=True)

        @pl.loop(0, W)
        def _zero(i):
            for q in range(COLS // LANES):
                window[i, pl.ds(q * LANES, LANES)] = jnp.zeros((LANES,), jnp.float32)

    def accum(h, n_idx, r, v, wb):
        roff = r - wb
        ab0 = gbufs[h, n_idx, pl.ds(0, 2 * LANES)]
        ab1 = gbufs[h, n_idx, pl.ds(2 * LANES, 2 * LANES)]
        a0, a1 = plsc.unpack(ab0, format=plsc.PackFormat.INTERLEAVED)
        a2, a3 = plsc.unpack(ab1, format=plsc.PackFormat.INTERLEAVED)
        for q, aq in enumerate((a0, a1, a2, a3)):
            sl = pl.ds(q * LANES, LANES)
            window[roff, sl] = window[roff, sl] + aq * v

    def direct(h, n_idx, r, v):
        fidx1[pl.ds(0, LANES)] = jnp.full((LANES,), r, jnp.int32)
        ab0 = gbufs[h, n_idx, pl.ds(0, 2 * LANES)]
        ab1 = gbufs[h, n_idx, pl.ds(2 * LANES, 2 * LANES)]
        a0, a1 = plsc.unpack(ab0, format=plsc.PackFormat.INTERLEAVED)
        a2, a3 = plsc.unpack(ab1, format=plsc.PackFormat.INTERLEAVED)
        for q, aq in enumerate((a0, a1, a2, a3)):
            cbuf[0, pl.ds(q * LANES, LANES)] = aq * v
        pltpu.sync_copy(cbuf, acc.at[fidx1], add=True)

    def unpack4(h, n_idx):
        ab0 = gbufs[h, n_idx, pl.ds(0, 2 * LANES)]
        ab1 = gbufs[h, n_idx, pl.ds(2 * LANES, 2 * LANES)]
        a0, a1 = plsc.unpack(ab0, format=plsc.PackFormat.INTERLEAVED)
        a2, a3 = plsc.unpack(ab1, format=plsc.PackFormat.INTERLEAVED)
        return (a0, a1, a2, a3)

    def compute_group(slot, h, g):
        @pl.loop(0, GB // LANES)
        def _sub(sg):
            rv = rows_v[slot, g, pl.ds(sg * LANES, LANES)]
            vv = vals_v[slot, g, pl.ds(sg * LANES, LANES)]
            r_f = rv[0]
            r_l = rv[LANES - 1]

            @pl.when(r_l >= wbase[0] + W)
            def _():
                flush_window()
                wbase[0] = r_f

            wb = wbase[0]

            @pl.when(r_f == r_l)
            def _run():
                s = [None] * (COLS // LANES)
                for i in range(0, LANES, 2):
                    n0 = sg * LANES + i
                    v0f = jnp.full((LANES,), vv[i], jnp.float32)
                    v1f = jnp.full((LANES,), vv[i + 1], jnp.float32)
                    v0 = plsc.pack(v0f, v0f, format=plsc.PackFormat.INTERLEAVED)
                    v1 = plsc.pack(v1f, v1f, format=plsc.PackFormat.INTERLEAVED)
                    for half in range(2):
                        hs = pl.ds(half * 2 * LANES, 2 * LANES)
                        p0 = gbufs[h, n0, hs] * v0
                        p1 = gbufs[h, n0 + 1, hs] * v1
                        qsum = p0 + p1
                        ua, ub = plsc.unpack(qsum, format=plsc.PackFormat.INTERLEAVED)
                        for k, u in ((2 * half, ua), (2 * half + 1, ub)):
                            s[k] = u if s[k] is None else s[k] + u
                roff = r_f - wb
                for q in range(COLS // LANES):
                    sl = pl.ds(q * LANES, LANES)
                    window[roff, sl] = window[roff, sl] + s[q]

            @pl.when((r_l < wb + W) & (r_f != r_l))
            def _fast():
                for i in range(LANES):
                    accum(h, sg * LANES + i, rv[i], vv[i], wb)

            @pl.when(r_l >= wb + W)
            def _wild():
                for i in range(LANES):
                    direct(h, sg * LANES + i, rv[i], vv[i])

    @pl.loop(0, W)
    def _zero_row(i):
        for q in range(COLS // LANES):
            window[i, pl.ds(q * LANES, LANES)] = jnp.zeros((LANES,), jnp.float32)

    @pl.loop(0, LANES)
    def _zero_cbuf(i):
        for q in range(COLS // LANES):
            cbuf[i, pl.ds(q * LANES, LANES)] = jnp.zeros((LANES,), jnp.float32)

    for k in range(ROWS_PER_TILE // W):
        pltpu.sync_copy(window, acc.at[pl.ds(sid * ROWS_PER_TILE + k * W, W)])

    @pl.when(sid == 0)
    def _():
        pltpu.sync_copy(window, acc.at[pl.ds(N, W)])

    pltpu.sync_copy(b_hbm.at[pl.ds(sid * ROWS_PER_TILE, ROWS_PER_TILE)],
                    bspm.at[pl.ds(sid * ROWS_PER_TILE, ROWS_PER_TILE)])
    plsc.subcore_barrier()

    idx_start(0, 0)
    idx_wait(0)
    rv0 = rows_v[0, 0, pl.ds(0, LANES)]
    wbase[0] = rv0[0]
    gathers_start(0, 0, 0)
    idx_start(1, 1)

    @pl.loop(0, n_chunks, step=2)
    def _cpair(ci0):
        for sc in range(2):
            ci = ci0 + sc

            @pl.loop(0, NGC, step=2)
            def _gpair(g0):
                for hh in range(2):
                    g = g0 + hh

                    @pl.when((g == NGC - 1) & (ci < n_chunks - 1))
                    def _():
                        idx_wait(1 - sc)

                    @pl.when(g < NGC - 1)
                    def _():
                        gathers_start(sc, 1 - hh, g + 1)

                    @pl.when((g == NGC - 1) & (ci < n_chunks - 1))
                    def _():
                        gathers_start(1 - sc, 1 - hh, 0)

                    @pl.when((g == 0) & (ci >= 1) & (ci < n_chunks - 1))
                    def _():
                        idx_start(ci + 1, 1 - sc)

                    gathers_wait(sc, hh, g)
                    compute_group(sc, hh, g)

    flush_window()
    plsc.subcore_barrier()
    pltpu.sync_copy(acc.at[pl.ds(sid * ROWS_PER_TILE, ROWS_PER_TILE)],
                    out_hbm.at[cid, pl.ds(sid * ROWS_PER_TILE, ROWS_PER_TILE)])


def _combine_body(p_ref, o_ref):
    o_ref[...] = p_ref[0] + p_ref[1]


def kernel(matrix_B, A_vals, A_rows, A_cols):
    nnz = A_vals.shape[0]
    per_w = ((nnz + NW * 2 * CHUNK - 1) // (NW * 2 * CHUNK)) * (2 * CHUNK)
    total = per_w * NW
    n_chunks = per_w // CHUNK
    pad = total - nnz

    b16 = matrix_B[:, _PERM].astype(jnp.bfloat16)
    cols = jnp.pad(A_cols.astype(jnp.int32), (0, pad)).reshape(total // GB, GB)
    rows = jnp.pad(A_rows.astype(jnp.int32), (0, pad),
                   constant_values=N - 1).reshape(total // GB, GB)
    vals = jnp.pad(A_vals, (0, pad)).reshape(total // GB, GB)

    mesh = plsc.VectorSubcoreMesh(core_axis_name="c", subcore_axis_name="s")
    partials = pl.kernel(
        functools.partial(_sc_body, n_chunks),
        out_type=jax.ShapeDtypeStruct((NC, N, COLS), jnp.float32),
        mesh=mesh,
        compiler_params=pltpu.CompilerParams(use_tc_tiling_on_sc=False,
                                             needs_layout_passes=False),
        scratch_types=[
            pltpu.VMEM_SHARED((N, COLS), jnp.bfloat16),
            pltpu.VMEM_SHARED((N + W, COLS), jnp.float32),
            pltpu.VMEM((W, COLS), jnp.float32),
            pltpu.VMEM((2, NGC, GB), jnp.int32),
            pltpu.VMEM((2, NGC, GB), jnp.int32),
            pltpu.VMEM((2, NGC, GB), jnp.float32),
            pltpu.VMEM((2, GB, COLS), jnp.bfloat16),
            pltpu.VMEM((128,), jnp.int32),
            pltpu.VMEM((LANES,), jnp.int32),
            pltpu.VMEM((LANES, COLS), jnp.float32),
            pltpu.SMEM((8,), jnp.int32),
            pltpu.SemaphoreType.DMA,
            pltpu.SemaphoreType.DMA,
            pltpu.SemaphoreType.DMA,
            pltpu.SemaphoreType.DMA,
        ],
    )(b16, vals, rows, cols)

    out = pl.pallas_call(
        _combine_body,
        out_shape=jax.ShapeDtypeStruct((N, COLS), jnp.float32),
        grid=(N // 1024,),
        in_specs=[pl.BlockSpec((NC, 1024, COLS), lambda i: (0, i, 0))],
        out_specs=pl.BlockSpec((1024, COLS), lambda i: (i, 0)),
    )(partials)
    return out


# speedup vs baseline: 4.2909x; 2.5095x over previous
import functools

import jax
import jax.numpy as jnp
import numpy as np
from jax import lax
from jax.experimental import pallas as pl
from jax.experimental.pallas import tpu as pltpu
from jax.experimental.pallas import tpu_sc as plsc

N = 16384
COLS = 64
NC = 2
NS = 16
NW = NC * NS
GB = 256
NGC = 4
CHUNK = NGC * GB
W = 128
ROWS_PER_TILE = N // NS
LANES = 16

_PERM = np.arange(COLS).reshape(COLS // 32, 2, 16).transpose(0, 2, 1).reshape(COLS)


def _sc_body(n_chunks, b_hbm, vals_hbm, rows_hbm, cols_hbm, out_hbm,
             bspm, acc, window, cols_v, rows_v, vals_v, gbufs, fidx, fidx1, cbuf, wbase,
             gsem0, gsem1, isem0, isem1):
    gsem = (gsem0, gsem1)
    isem = (isem0, isem1)
    cid = lax.axis_index("c")
    sid = lax.axis_index("s")
    wid = sid * NC + cid
    iota16 = lax.iota(jnp.int32, LANES)

    def idx_start(ci, slot):
        row0 = (wid * n_chunks + ci) * NGC
        pltpu.async_copy(cols_hbm.at[pl.ds(row0, NGC)], cols_v.at[slot], isem[slot])
        pltpu.async_copy(rows_hbm.at[pl.ds(row0, NGC)], rows_v.at[slot], isem[slot])
        pltpu.async_copy(vals_hbm.at[pl.ds(row0, NGC)], vals_v.at[slot], isem[slot])

    def idx_wait(slot):
        pltpu.make_async_copy(cols_hbm.at[pl.ds(0, NGC)], cols_v.at[slot], isem[slot]).wait()
        pltpu.make_async_copy(rows_hbm.at[pl.ds(0, NGC)], rows_v.at[slot], isem[slot]).wait()
        pltpu.make_async_copy(vals_hbm.at[pl.ds(0, NGC)], vals_v.at[slot], isem[slot]).wait()

    def gathers_start(slot, h, g):
        pltpu.async_copy(bspm.at[cols_v.at[slot, g]], gbufs.at[h], gsem[h])

    def gathers_wait(slot, h, g):
        pltpu.make_async_copy(bspm.at[cols_v.at[slot, g]], gbufs.at[h], gsem[h]).wait()

    def flush_window():
        wb = wbase[0]
        for fb in range(W // 128):
            @pl.loop(0, 128 // LANES)
            def _fill(t):
                fidx[pl.ds(t * LANES, LANES)] = (wb + fb * 128) + t * LANES + iota16

            pltpu.sync_copy(window.at[pl.ds(fb * 128, 128)], acc.at[fidx], add=True)

        @pl.loop(0, W)
        def _zero(i):
            for q in range(COLS // LANES):
                window[i, pl.ds(q * LANES, LANES)] = jnp.zeros((LANES,), jnp.float32)

    def accum(h, n_idx, r, v, wb):
        roff = r - wb
        ab0 = gbufs[h, n_idx, pl.ds(0, 2 * LANES)]
        ab1 = gbufs[h, n_idx, pl.ds(2 * LANES, 2 * LANES)]
        a0, a1 = plsc.unpack(ab0, format=plsc.PackFormat.INTERLEAVED)
        a2, a3 = plsc.unpack(ab1, format=plsc.PackFormat.INTERLEAVED)
        for q, aq in enumerate((a0, a1, a2, a3)):
            sl = pl.ds(q * LANES, LANES)
            window[roff, sl] = window[roff, sl] + aq * v

    def direct(h, n_idx, r, v):
        fidx1[pl.ds(0, LANES)] = jnp.full((LANES,), r, jnp.int32)
        ab0 = gbufs[h, n_idx, pl.ds(0, 2 * LANES)]
        ab1 = gbufs[h, n_idx, pl.ds(2 * LANES, 2 * LANES)]
        a0, a1 = plsc.unpack(ab0, format=plsc.PackFormat.INTERLEAVED)
        a2, a3 = plsc.unpack(ab1, format=plsc.PackFormat.INTERLEAVED)
        for q, aq in enumerate((a0, a1, a2, a3)):
            cbuf[0, pl.ds(q * LANES, LANES)] = aq * v
        pltpu.sync_copy(cbuf, acc.at[fidx1], add=True)

    def unpack4(h, n_idx):
        ab0 = gbufs[h, n_idx, pl.ds(0, 2 * LANES)]
        ab1 = gbufs[h, n_idx, pl.ds(2 * LANES, 2 * LANES)]
        a0, a1 = plsc.unpack(ab0, format=plsc.PackFormat.INTERLEAVED)
        a2, a3 = plsc.unpack(ab1, format=plsc.PackFormat.INTERLEAVED)
        return (a0, a1, a2, a3)

    def compute_group(slot, h, g):
        @functools.partial(plsc.parallel_loop, 0, GB // LANES, unroll=2)
        def _sub(sg):
            rv = rows_v[slot, g, pl.ds(sg * LANES, LANES)]
            vv = vals_v[slot, g, pl.ds(sg * LANES, LANES)]
            r_f = rv[0]
            r_l = rv[LANES - 1]

            @pl.when(r_l >= wbase[0] + W)
            def _():
                flush_window()
                wbase[0] = r_f

            wb = wbase[0]

            @pl.when(r_f == r_l)
            def _run():
                s = [None] * (COLS // LANES)
                for i in range(0, LANES, 2):
                    n0 = sg * LANES + i
                    v0f = jnp.full((LANES,), vv[i], jnp.float32)
                    v1f = jnp.full((LANES,), vv[i + 1], jnp.float32)
                    v0 = plsc.pack(v0f, v0f, format=plsc.PackFormat.INTERLEAVED)
                    v1 = plsc.pack(v1f, v1f, format=plsc.PackFormat.INTERLEAVED)
                    for half in range(2):
                        hs = pl.ds(half * 2 * LANES, 2 * LANES)
                        p0 = gbufs[h, n0, hs] * v0
                        p1 = gbufs[h, n0 + 1, hs] * v1
                        qsum = p0 + p1
                        ua, ub = plsc.unpack(qsum, format=plsc.PackFormat.INTERLEAVED)
                        for k, u in ((2 * half, ua), (2 * half + 1, ub)):
                            s[k] = u if s[k] is None else s[k] + u
                roff = r_f - wb
                for q in range(COLS // LANES):
                    sl = pl.ds(q * LANES, LANES)
                    window[roff, sl] = window[roff, sl] + s[q]

            @pl.when((r_l < wb + W) & (r_f != r_l))
            def _fast():
                for i in range(LANES):
                    accum(h, sg * LANES + i, rv[i], vv[i], wb)

            @pl.when(r_l >= wb + W)
            def _wild():
                for i in range(LANES):
                    direct(h, sg * LANES + i, rv[i], vv[i])

    @pl.loop(0, W)
    def _zero_row(i):
        for q in range(COLS // LANES):
            window[i, pl.ds(q * LANES, LANES)] = jnp.zeros((LANES,), jnp.float32)

    @pl.loop(0, LANES)
    def _zero_cbuf(i):
        for q in range(COLS // LANES):
            cbuf[i, pl.ds(q * LANES, LANES)] = jnp.zeros((LANES,), jnp.float32)

    for k in range(ROWS_PER_TILE // W):
        pltpu.sync_copy(window, acc.at[pl.ds(sid * ROWS_PER_TILE + k * W, W)])

    @pl.when(sid == 0)
    def _():
        pltpu.sync_copy(window, acc.at[pl.ds(N, W)])

    pltpu.sync_copy(b_hbm.at[pl.ds(sid * ROWS_PER_TILE, ROWS_PER_TILE)],
                    bspm.at[pl.ds(sid * ROWS_PER_TILE, ROWS_PER_TILE)])
    plsc.subcore_barrier()

    idx_start(0, 0)
    idx_wait(0)
    rv0 = rows_v[0, 0, pl.ds(0, LANES)]
    wbase[0] = rv0[0]
    gathers_start(0, 0, 0)
    idx_start(1, 1)

    @pl.loop(0, n_chunks, step=2)
    def _cpair(ci0):
        for sc in range(2):
            ci = ci0 + sc

            @pl.loop(0, NGC, step=2)
            def _gpair(g0):
                for hh in range(2):
                    g = g0 + hh

                    @pl.when((g == NGC - 1) & (ci < n_chunks - 1))
                    def _():
                        idx_wait(1 - sc)

                    @pl.when(g < NGC - 1)
                    def _():
                        gathers_start(sc, 1 - hh, g + 1)

                    @pl.when((g == NGC - 1) & (ci < n_chunks - 1))
                    def _():
                        gathers_start(1 - sc, 1 - hh, 0)

                    @pl.when((g == 0) & (ci >= 1) & (ci < n_chunks - 1))
                    def _():
                        idx_start(ci + 1, 1 - sc)

                    gathers_wait(sc, hh, g)
                    compute_group(sc, hh, g)

    flush_window()
    plsc.subcore_barrier()
    pltpu.sync_copy(acc.at[pl.ds(sid * ROWS_PER_TILE, ROWS_PER_TILE)],
                    out_hbm.at[cid, pl.ds(sid * ROWS_PER_TILE, ROWS_PER_TILE)])


def _combine_body(p_ref, o_ref):
    o_ref[...] = p_ref[0] + p_ref[1]


def kernel(matrix_B, A_vals, A_rows, A_cols):
    nnz = A_vals.shape[0]
    per_w = ((nnz + NW * 2 * CHUNK - 1) // (NW * 2 * CHUNK)) * (2 * CHUNK)
    total = per_w * NW
    n_chunks = per_w // CHUNK
    pad = total - nnz

    b16 = matrix_B[:, _PERM].astype(jnp.bfloat16)
    cols = jnp.pad(A_cols.astype(jnp.int32), (0, pad)).reshape(total // GB, GB)
    rows = jnp.pad(A_rows.astype(jnp.int32), (0, pad),
                   constant_values=N - 1).reshape(total // GB, GB)
    vals = jnp.pad(A_vals, (0, pad)).reshape(total // GB, GB)

    mesh = plsc.VectorSubcoreMesh(core_axis_name="c", subcore_axis_name="s")
    partials = pl.kernel(
        functools.partial(_sc_body, n_chunks),
        out_type=jax.ShapeDtypeStruct((NC, N, COLS), jnp.float32),
        mesh=mesh,
        compiler_params=pltpu.CompilerParams(use_tc_tiling_on_sc=False,
                                             needs_layout_passes=False),
        scratch_types=[
            pltpu.VMEM_SHARED((N, COLS), jnp.bfloat16),
            pltpu.VMEM_SHARED((N + W, COLS), jnp.float32),
            pltpu.VMEM((W, COLS), jnp.float32),
            pltpu.VMEM((2, NGC, GB), jnp.int32),
            pltpu.VMEM((2, NGC, GB), jnp.int32),
            pltpu.VMEM((2, NGC, GB), jnp.float32),
            pltpu.VMEM((2, GB, COLS), jnp.bfloat16),
            pltpu.VMEM((128,), jnp.int32),
            pltpu.VMEM((LANES,), jnp.int32),
            pltpu.VMEM((LANES, COLS), jnp.float32),
            pltpu.SMEM((8,), jnp.int32),
            pltpu.SemaphoreType.DMA,
            pltpu.SemaphoreType.DMA,
            pltpu.SemaphoreType.DMA,
            pltpu.SemaphoreType.DMA,
        ],
    )(b16, vals, rows, cols)

    out = pl.pallas_call(
        _combine_body,
        out_shape=jax.ShapeDtypeStruct((N, COLS), jnp.float32),
        grid=(N // 1024,),
        in_specs=[pl.BlockSpec((NC, 1024, COLS), lambda i: (0, i, 0))],
        out_specs=pl.BlockSpec((1024, COLS), lambda i: (i, 0)),
    )(partials)
    return out
